# baseline (speedup 1.0000x reference)
import jax
import jax.numpy as jnp
from jax import lax
from jax.experimental import pallas as pl

_T, _T2, _B, _D, _M = 2000, 1635, 1000, 35, 4
_K = 8
_L = 5000
_RB = _L // _K
_LANES = _K * _D
_NBLK = (_T2 * _B) // _L
_OFF = ((_T - _T2) * _B) // _L
_NXC = (_T * _B) // _L


def _ens_kernel(xc_ref, p0_ref, p1_ref, p2_ref, p3_ref, w_ref, b_ref, out_ref):
    x = xc_ref[0]
    logits = lax.dot_general(
        x, w_ref[...], (((1,), (0,)), ((), ())),
        preferred_element_type=jnp.float32)
    logits = logits + b_ref[...]
    l0 = logits[:, 0 * _K:1 * _K]
    l1 = logits[:, 1 * _K:2 * _K]
    l2 = logits[:, 2 * _K:3 * _K]
    l3 = logits[:, 3 * _K:4 * _K]
    p01 = jnp.where(l0 >= l1, p0_ref[0], p1_ref[0])
    v01 = jnp.maximum(l0, l1)
    p23 = jnp.where(l2 >= l3, p2_ref[0], p3_ref[0])
    v23 = jnp.maximum(l2, l3)
    out_ref[0] = jnp.where(v01 >= v23, p01, p23)


def kernel(xc_nn_norm, target, pred_m0, pred_m1, pred_m2, pred_m3, W, b):
    del target
    xc = xc_nn_norm.reshape(_NXC, _RB, _LANES)
    p0 = pred_m0.reshape(_NBLK, _RB, _K)
    p1 = pred_m1.reshape(_NBLK, _RB, _K)
    p2 = pred_m2.reshape(_NBLK, _RB, _K)
    p3 = pred_m3.reshape(_NBLK, _RB, _K)
    wbig = jnp.einsum('dm,jk->jdmk', W, jnp.eye(_K, dtype=W.dtype))
    wbig = wbig.reshape(_K * _D, _M * _K)
    bbig = jnp.repeat(b, _K).reshape(1, _M * _K)
    pspec = pl.BlockSpec((1, _RB, _K), lambda i: (i, 0, 0))
    out = pl.pallas_call(
        _ens_kernel,
        grid=(_NBLK,),
        in_specs=[
            pl.BlockSpec((1, _RB, _LANES), lambda i: (i + _OFF, 0, 0)),
            pspec, pspec, pspec, pspec,
            pl.BlockSpec((_K * _D, _M * _K), lambda i: (0, 0)),
            pl.BlockSpec((1, _M * _K), lambda i: (0, 0)),
        ],
        out_specs=pl.BlockSpec((1, _RB, _K), lambda i: (i, 0, 0)),
        out_shape=jax.ShapeDtypeStruct((_NBLK, _RB, _K), jnp.float32),
    )(xc, p0, p1, p2, p3, wbig, bbig)
    return out.reshape(_T2, _B)


# speedup vs baseline: 1.3600x; 1.3600x over previous
import jax
import jax.numpy as jnp
from jax import lax
from jax.experimental import pallas as pl

_T, _T2, _B, _D, _M = 2000, 1635, 1000, 35, 4
_R = 5000
_NBLK = (_T2 * _B) // _R
_OFF = ((_T - _T2) * _B) // _R


def _ens_kernel(xc_ref, p0_ref, p1_ref, p2_ref, p3_ref, w_ref, b_ref, out_ref):
    x = xc_ref[...]
    w = w_ref[...]
    logits = lax.dot_general(
        w, x, (((0,), (1,)), ((), ())),
        preferred_element_type=jnp.float32)
    b = b_ref[...]
    l0 = logits[0:1, :] + b[0, 0]
    l1 = logits[1:2, :] + b[0, 1]
    l2 = logits[2:3, :] + b[0, 2]
    l3 = logits[3:4, :] + b[0, 3]
    p01 = jnp.where(l0 >= l1, p0_ref[0], p1_ref[0])
    v01 = jnp.maximum(l0, l1)
    p23 = jnp.where(l2 >= l3, p2_ref[0], p3_ref[0])
    v23 = jnp.maximum(l2, l3)
    out_ref[0] = jnp.where(v01 >= v23, p01, p23)


def kernel(xc_nn_norm, target, pred_m0, pred_m1, pred_m2, pred_m3, W, b):
    del target
    xc = xc_nn_norm.reshape(_T * _B, _D)
    p0 = pred_m0.reshape(_NBLK, 1, _R)
    p1 = pred_m1.reshape(_NBLK, 1, _R)
    p2 = pred_m2.reshape(_NBLK, 1, _R)
    p3 = pred_m3.reshape(_NBLK, 1, _R)
    pspec = pl.BlockSpec((1, 1, _R), lambda i: (i, 0, 0))
    out = pl.pallas_call(
        _ens_kernel,
        grid=(_NBLK,),
        in_specs=[
            pl.BlockSpec((_R, _D), lambda i: (i + _OFF, 0)),
            pspec, pspec, pspec, pspec,
            pl.BlockSpec((_D, _M), lambda i: (0, 0)),
            pl.BlockSpec((1, _M), lambda i: (0, 0)),
        ],
        out_specs=pl.BlockSpec((1, 1, _R), lambda i: (i, 0, 0)),
        out_shape=jax.ShapeDtypeStruct((_NBLK, 1, _R), jnp.float32),
    )(xc, p0, p1, p2, p3, W, b.reshape(1, _M))
    return out.reshape(_T2, _B)


# speedup vs baseline: 2.0101x; 1.4780x over previous
import jax
import jax.numpy as jnp
from jax import lax
from jax.experimental import pallas as pl

_T, _T2, _B, _D, _M = 2000, 1635, 1000, 35, 4
_R = 5000
_NBLK = (_T2 * _B) // _R
_OFF = ((_T - _T2) * _B) // _R


def _ens_kernel(xc_ref, p0_ref, p1_ref, p2_ref, p3_ref, w_ref, b_ref, out_ref):
    x = xc_ref[...].reshape(_R, _D)
    w = w_ref[...]
    logits = lax.dot_general(
        w, x, (((0,), (1,)), ((), ())),
        preferred_element_type=jnp.float32)
    b = b_ref[...]
    l0 = logits[0:1, :] + b[0, 0]
    l1 = logits[1:2, :] + b[0, 1]
    l2 = logits[2:3, :] + b[0, 2]
    l3 = logits[3:4, :] + b[0, 3]
    p01 = jnp.where(l0 >= l1, p0_ref[0], p1_ref[0])
    v01 = jnp.maximum(l0, l1)
    p23 = jnp.where(l2 >= l3, p2_ref[0], p3_ref[0])
    v23 = jnp.maximum(l2, l3)
    out_ref[0] = jnp.where(v01 >= v23, p01, p23)


def kernel(xc_nn_norm, target, pred_m0, pred_m1, pred_m2, pred_m3, W, b):
    del target
    p0 = pred_m0.reshape(_NBLK, 1, _R)
    p1 = pred_m1.reshape(_NBLK, 1, _R)
    p2 = pred_m2.reshape(_NBLK, 1, _R)
    p3 = pred_m3.reshape(_NBLK, 1, _R)
    pspec = pl.BlockSpec((1, 1, _R), lambda i: (i, 0, 0))
    out = pl.pallas_call(
        _ens_kernel,
        grid=(_NBLK,),
        in_specs=[
            pl.BlockSpec((_R // _B, _B, _D), lambda i: (i + _OFF, 0, 0)),
            pspec, pspec, pspec, pspec,
            pl.BlockSpec((_D, _M), lambda i: (0, 0)),
            pl.BlockSpec((1, _M), lambda i: (0, 0)),
        ],
        out_specs=pl.BlockSpec((1, 1, _R), lambda i: (i, 0, 0)),
        out_shape=jax.ShapeDtypeStruct((_NBLK, 1, _R), jnp.float32),
    )(xc_nn_norm, p0, p1, p2, p3, W, b.reshape(1, _M))
    return out.reshape(_T2, _B)


# speedup vs baseline: 2.2575x; 1.1231x over previous
import jax
import jax.numpy as jnp
from jax import lax
from jax.experimental import pallas as pl

_T, _T2, _B, _D, _M = 2000, 1635, 1000, 35, 4
_TT = 5
_R = _TT * _B
_NCH = _T2 // _TT
_OFF = (_T - _T2) // _TT
_NSPLIT = 3
_NBLK = _NCH // _NSPLIT


def _ens_kernel(xc0_ref, xc1_ref, xc2_ref,
                p0_ref, p1_ref, p2_ref, p3_ref, w_ref, b_ref, out_ref):
    w = w_ref[...]
    b = b_ref[...]
    for k, xref in enumerate((xc0_ref, xc1_ref, xc2_ref)):
        x = xref[...].reshape(_R, _D)
        logits = lax.dot_general(
            w, x, (((0,), (1,)), ((), ())),
            preferred_element_type=jnp.float32)
        l0 = logits[0:1, :] + b[0, 0]
        l1 = logits[1:2, :] + b[0, 1]
        l2 = logits[2:3, :] + b[0, 2]
        l3 = logits[3:4, :] + b[0, 3]
        p01 = jnp.where(l0 >= l1, p0_ref[k], p1_ref[k])
        v01 = jnp.maximum(l0, l1)
        p23 = jnp.where(l2 >= l3, p2_ref[k], p3_ref[k])
        v23 = jnp.maximum(l2, l3)
        out_ref[k] = jnp.where(v01 >= v23, p01, p23)


def kernel(xc_nn_norm, target, pred_m0, pred_m1, pred_m2, pred_m3, W, b):
    del target
    p0 = pred_m0.reshape(_NCH, 1, _R)
    p1 = pred_m1.reshape(_NCH, 1, _R)
    p2 = pred_m2.reshape(_NCH, 1, _R)
    p3 = pred_m3.reshape(_NCH, 1, _R)
    pspec = pl.BlockSpec((_NSPLIT, 1, _R), lambda i: (i, 0, 0))
    xcspecs = [
        pl.BlockSpec((_TT, _B, _D),
                     lambda i, k=k: (_NSPLIT * i + _OFF + k, 0, 0))
        for k in range(_NSPLIT)
    ]
    out = pl.pallas_call(
        _ens_kernel,
        grid=(_NBLK,),
        in_specs=xcspecs + [
            pspec, pspec, pspec, pspec,
            pl.BlockSpec((_D, _M), lambda i: (0, 0)),
            pl.BlockSpec((1, _M), lambda i: (0, 0)),
        ],
        out_specs=pl.BlockSpec((_NSPLIT, 1, _R), lambda i: (i, 0, 0)),
        out_shape=jax.ShapeDtypeStruct((_NCH, 1, _R), jnp.float32),
    )(xc_nn_norm, xc_nn_norm, xc_nn_norm, p0, p1, p2, p3, W, b.reshape(1, _M))
    return out.reshape(_T2, _B)


# speedup vs baseline: 8.4833x; 3.7578x over previous
import jax
import jax.numpy as jnp
from jax.experimental import pallas as pl
from jax.experimental.pallas import tpu as pltpu

_T, _T2, _B, _D, _M = 2000, 1635, 1000, 35, 4
_T0 = 360
_TT = 40
_NBLK = (_T - _T0) // _TT
_OFFB = _T0 // _TT


def _ens_kernel(xc_ref, p0_ref, p1_ref, p2_ref, p3_ref, w_ref, b_ref, out_ref):
    x = xc_ref[...]
    ls = []
    for m in range(_M):
        acc = x[0] * w_ref[0, m]
        for d in range(1, _D):
            acc = acc + x[d] * w_ref[d, m]
        ls.append(acc + b_ref[m])
    l0, l1, l2, l3 = ls
    p01 = jnp.where(l0 >= l1, p0_ref[...], p1_ref[...])
    v01 = jnp.maximum(l0, l1)
    p23 = jnp.where(l2 >= l3, p2_ref[...], p3_ref[...])
    v23 = jnp.maximum(l2, l3)
    out_ref[...] = jnp.where(v01 >= v23, p01, p23)


def kernel(xc_nn_norm, target, pred_m0, pred_m1, pred_m2, pred_m3, W, b):
    del target
    xc_t = jnp.transpose(xc_nn_norm, (2, 0, 1))
    pad = (_T - _T0) - _T2
    pp = [
        jnp.pad(p.reshape(_T2, _B), ((pad, 0), (0, 0)))
        for p in (pred_m0, pred_m1, pred_m2, pred_m3)
    ]
    pspec = pl.BlockSpec((_TT, _B), lambda i: (i, 0))
    out = pl.pallas_call(
        _ens_kernel,
        grid=(_NBLK,),
        in_specs=[
            pl.BlockSpec((_D, _TT, _B), lambda i: (0, i + _OFFB, 0)),
            pspec, pspec, pspec, pspec,
            pl.BlockSpec(memory_space=pltpu.SMEM),
            pl.BlockSpec(memory_space=pltpu.SMEM),
        ],
        out_specs=pl.BlockSpec((_TT, _B), lambda i: (i, 0)),
        out_shape=jax.ShapeDtypeStruct((_T2 + pad, _B), jnp.float32),
    )(xc_t, *pp, W, b)
    return out[pad:]
